# baseline (speedup 1.0000x reference)
import functools

import jax
import jax.numpy as jnp
from jax.experimental import pallas as pl
from jax.experimental.pallas import tpu as pltpu

_EPS = 1e-5
_BLOCK_ROWS = 2048


def _ln_add_kernel(idx_ref, x_ref, tab_ref, g_ref, b_ref, o_ref):
    x = x_ref[...]
    mean = jnp.mean(x, axis=-1, keepdims=True)
    msq = jnp.mean(x * x, axis=-1, keepdims=True)
    var = jnp.maximum(msq - mean * mean, 0.0)
    scale = jax.lax.rsqrt(var + _EPS)
    emb = tab_ref[idx_ref[0], :]
    o_ref[...] = (x - mean) * (scale * g_ref[...]) + (b_ref[...] + emb)


@functools.partial(jax.jit, static_argnames=("interpret",))
def _run(x, table, gamma, beta, modality_idx, interpret=False):
    orig_shape = x.shape
    dim = orig_shape[-1]
    x2 = x.reshape(-1, dim)
    rows = x2.shape[0]
    br = min(_BLOCK_ROWS, rows)
    grid = (rows // br,)
    out = pl.pallas_call(
        _ln_add_kernel,
        grid_spec=pltpu.PrefetchScalarGridSpec(
            num_scalar_prefetch=1,
            grid=grid,
            in_specs=[
                pl.BlockSpec((br, dim), lambda i, *_: (i, 0)),
                pl.BlockSpec(table.shape, lambda i, *_: (0, 0)),
                pl.BlockSpec((1, dim), lambda i, *_: (0, 0)),
                pl.BlockSpec((1, dim), lambda i, *_: (0, 0)),
            ],
            out_specs=pl.BlockSpec((br, dim), lambda i, *_: (i, 0)),
        ),
        out_shape=jax.ShapeDtypeStruct((rows, dim), x.dtype),
        compiler_params=pltpu.CompilerParams(
            dimension_semantics=("parallel",),
        ),
        interpret=interpret,
    )(
        modality_idx.reshape(1).astype(jnp.int32),
        x2,
        table,
        gamma.reshape(1, dim),
        beta.reshape(1, dim),
    )
    return out.reshape(orig_shape)


def kernel(x, table, gamma, beta, modality_idx):
    return _run(x, table, gamma, beta, modality_idx)


# speedup vs baseline: 1.0147x; 1.0147x over previous
import functools

import jax
import jax.numpy as jnp
from jax.experimental import pallas as pl
from jax.experimental.pallas import tpu as pltpu

_EPS = 1e-5
_BLOCK_ROWS = 3280


def _ln_add_kernel(idx_ref, x_ref, tab_ref, g_ref, b_ref, o_ref):
    x = x_ref[...]
    mean = jnp.mean(x, axis=-1, keepdims=True)
    c = x - mean
    var = jnp.mean(c * c, axis=-1, keepdims=True)
    xn = c * jax.lax.rsqrt(var + _EPS)
    emb = tab_ref[idx_ref[0], :]
    o_ref[...] = xn * g_ref[...] + (b_ref[...] + emb)


@functools.partial(jax.jit, static_argnames=("interpret",))
def _run(x, table, gamma, beta, modality_idx, interpret=False):
    orig_shape = x.shape
    dim = orig_shape[-1]
    x2 = x.reshape(-1, dim)
    rows = x2.shape[0]
    br = min(_BLOCK_ROWS, rows)
    grid = (pl.cdiv(rows, br),)
    out = pl.pallas_call(
        _ln_add_kernel,
        grid_spec=pltpu.PrefetchScalarGridSpec(
            num_scalar_prefetch=1,
            grid=grid,
            in_specs=[
                pl.BlockSpec((br, dim), lambda i, *_: (i, 0)),
                pl.BlockSpec(table.shape, lambda i, *_: (0, 0)),
                pl.BlockSpec((1, dim), lambda i, *_: (0, 0)),
                pl.BlockSpec((1, dim), lambda i, *_: (0, 0)),
            ],
            out_specs=pl.BlockSpec((br, dim), lambda i, *_: (i, 0)),
        ),
        out_shape=jax.ShapeDtypeStruct((rows, dim), x.dtype),
        compiler_params=pltpu.CompilerParams(
            dimension_semantics=("parallel",),
        ),
        interpret=interpret,
    )(
        modality_idx.reshape(1).astype(jnp.int32),
        x2,
        table,
        gamma.reshape(1, dim),
        beta.reshape(1, dim),
    )
    return out.reshape(orig_shape)


def kernel(x, table, gamma, beta, modality_idx):
    return _run(x, table, gamma, beta, modality_idx)
